# Initial kernel scaffold; baseline (speedup 1.0000x reference)
import jax
import jax.numpy as jnp
from jax.experimental import pallas as pl


def kernel(x, W1, b1, W2, b2, codebook):
    raise NotImplementedError("write your pallas kernel here")


# speedup vs baseline: 1.0973x; 1.0973x over previous
import functools

import jax
import jax.numpy as jnp
from jax import lax
from jax.experimental import pallas as pl
from jax.experimental.pallas import tpu as pltpu
from jax.experimental.pallas import tpu_sc as plsc

B, T, D_IN, D_H, K = 16, 1024, 256, 64, 8192
N = B * T

R = 256
CB_CHUNK = 2048


def _encode_argmin_body(x_ref, w1_ref, b1_ref, w2_ref, b2_ref, cb_ref, idx_ref):
    x = x_ref[...]
    h = jnp.tanh(
        jnp.dot(x, w1_ref[...], preferred_element_type=jnp.float32)
        + b1_ref[...])
    hb = h.astype(jnp.bfloat16).astype(jnp.float32)
    z = (jnp.dot(hb, w2_ref[...], preferred_element_type=jnp.float32)
         + b2_ref[...])
    zn = jnp.sum(z * z, axis=1, keepdims=True)
    zb = z.astype(jnp.bfloat16).astype(jnp.float32)

    best_d = jnp.full((R,), jnp.inf, dtype=jnp.float32)
    best_i = jnp.zeros((R,), dtype=jnp.int32)
    ones_row = jnp.ones((1, D_H), dtype=jnp.float32)
    for c in range(0, K, CB_CHUNK):
        cb = cb_ref[pl.ds(c, CB_CHUNK), :]
        cbn = lax.dot_general(ones_row, cb * cb,
                              (((1,), (1,)), ((), ())),
                              preferred_element_type=jnp.float32)
        sc = lax.dot_general(zb, cb, (((1,), (1,)), ((), ())),
                             preferred_element_type=jnp.float32)
        d = zn - 2.0 * sc + cbn
        m = jnp.min(d, axis=1, keepdims=True)
        ii = lax.broadcasted_iota(jnp.int32, (R, CB_CHUNK), 1) + c
        li = jnp.min(jnp.where(d <= m, ii, jnp.int32(2**30)), axis=1)
        mrow = m[:, 0]
        upd = mrow < best_d
        best_i = jnp.where(upd, li, best_i)
        best_d = jnp.where(upd, mrow, best_d)

    idx_ref[...] = best_i.reshape(1, 1, R)


def _encode_argmin(x2d, W1, b1, W2, b2, codebook):
    grid = (N // R,)
    out = pl.pallas_call(
        _encode_argmin_body,
        grid=grid,
        in_specs=[
            pl.BlockSpec((R, D_IN), lambda i: (i, 0)),
            pl.BlockSpec((D_IN, D_H), lambda i: (0, 0)),
            pl.BlockSpec((1, D_H), lambda i: (0, 0)),
            pl.BlockSpec((D_H, D_H), lambda i: (0, 0)),
            pl.BlockSpec((1, D_H), lambda i: (0, 0)),
            pl.BlockSpec((K, D_H), lambda i: (0, 0)),
        ],
        out_specs=pl.BlockSpec((1, 1, R), lambda i: (i, 0, 0)),
        out_shape=jax.ShapeDtypeStruct((N // R, 1, R), jnp.int32),
    )(x2d, W1, b1.reshape(1, D_H), W2, b2.reshape(1, D_H), codebook)
    return out.reshape(N)


_NC, _NS = 2, 16
_NW = _NC * _NS
_BPW = N // _NW
_DPAD = 128


def _sc_gather_body(table_hbm, idx_hbm, out_hbm, idx_v, rows_v, sem):
    wid = lax.axis_index("s") * _NC + lax.axis_index("c")
    base = wid * _BPW
    pltpu.sync_copy(idx_hbm.at[pl.ds(base, _BPW)], idx_v)
    pltpu.async_copy(table_hbm.at[idx_v], rows_v, sem).wait()
    pltpu.sync_copy(rows_v, out_hbm.at[pl.ds(base, _BPW)])


def _sc_gather(codebook_padded, idx):
    mesh = plsc.VectorSubcoreMesh(core_axis_name="c", subcore_axis_name="s",
                                  num_cores=_NC)
    f = functools.partial(
        pl.kernel,
        mesh=mesh,
        out_type=jax.ShapeDtypeStruct((N, _DPAD), jnp.float32),
        scratch_types=[
            pltpu.VMEM((_BPW,), jnp.int32),
            pltpu.VMEM((_BPW, _DPAD), jnp.float32),
            pltpu.SemaphoreType.DMA,
        ],
    )(_sc_gather_body)
    return f(codebook_padded, idx)


def kernel(x, W1, b1, W2, b2, codebook):
    x2d = x.reshape(N, D_IN)
    z_id = _encode_argmin(x2d, W1, b1, W2, b2, codebook)
    cb_pad = jnp.pad(codebook, ((0, 0), (0, _DPAD - D_H)))
    z_q = _sc_gather(cb_pad, z_id)[:, :D_H]
    return z_q.reshape(B, T, D_H), z_id.reshape(B, T)
